# baseline (speedup 1.0000x reference)
import jax
import jax.numpy as jnp
import numpy as np
from jax.experimental import pallas as pl
from jax.experimental.pallas import tpu as pltpu

N = 2048
MOL = 32
B = 64
K = 28
NF = 256
NG = 50
NI = 4
CUTOFF = 6.0

MB = 4
BLKA = MB * MOL
NBLK = N // BLKA
E_BLK = BLKA * MOL

_OFF_STEP = np.float32(CUTOFF / (NG - 1))
_COEFF = np.float32(-0.5 / _OFF_STEP ** 2)
_LOG2 = np.float32(np.log(2.0))
_NDROP = MOL - 1 - K


def _sp(x):
    return jax.nn.softplus(x) - _LOG2


def _dotg(a, b, dims):
    return jax.lax.dot_general(a, b, (dims, ((), ())))


def _iota(shape, dim):
    return jax.lax.broadcasted_iota(jnp.int32, shape, dim)


def _schnet_kernel(pos_ref, z_ref, af_ref, mf_ref, sm_ref, eye_ref,
                   emb_ref, w1_ref, b1_ref, w2_ref, b2_ref,
                   cw1_ref, cw2_ref, cb2_ref, bw_ref, bb_ref,
                   l1w_ref, l1b_ref, l2w_ref, l2b_ref, out_ref):
    f32 = jnp.float32
    p = pos_ref[...]
    zrow = z_ref[0]

    ones3 = jnp.ones((MOL, 3), f32)
    ones1 = jnp.ones((MOL, 1), f32)
    ew_rows = []
    cm_rows = []
    for mm in range(MB):
        pm = p[mm * MOL:(mm + 1) * MOL, :]
        pp = pm * pm
        sq_col = _dotg(pp, ones3, ((1,), (1,)))
        sq_row = _dotg(ones3, pp, ((1,), (1,)))
        cross = _dotg(pm, pm, ((1,), (1,)))
        d2 = sq_col + sq_row - 2.0 * cross

        di = _iota((MOL, MOL), 0)
        si = _iota((MOL, MOL), 1)
        valid = di != si
        for _ in range(_NDROP):
            vals = jnp.where(valid, d2, -1e30)
            rmax = jnp.max(vals, axis=1, keepdims=True)
            cand = valid & (vals >= rmax)
            jmax = jnp.max(jnp.where(cand, si, -1), axis=1, keepdims=True)
            valid = valid & (si != jmax)

        acc = jnp.zeros((MOL, MOL), f32)
        for c in range(3):
            pc = pm[:, c:c + 1]
            pc_row = _dotg(ones1, pc, ((1,), (1,)))
            dif = pc - pc_row
            acc = acc + dif * dif
        ew = jnp.sqrt(acc)
        ccut = 0.5 * (jnp.cos(ew * np.pi / CUTOFF) + 1.0)
        cm_rows.append(jnp.where(valid, ccut, 0.0))
        ew_rows.append(ew)

    ew_all = jnp.concatenate(ew_rows, axis=0)
    cm_all = jnp.concatenate(cm_rows, axis=0)

    af = af_ref[...]
    mf = mf_ref[...]
    ew_sel = _dotg(af, ew_all, ((1,), (0,))) * mf
    ew_col = jnp.sum(ew_sel, axis=1, keepdims=True)
    cm_sel = _dotg(af, cm_all, ((1,), (0,))) * mf
    cm_col = jnp.sum(cm_sel, axis=1, keepdims=True)

    off = (_iota((1, NG), 1).astype(f32)) * _OFF_STEP
    ea = jnp.exp(_COEFF * (ew_col - off) ** 2)

    onehot_t = (_iota((100, BLKA), 0) == zrow).astype(f32)
    h = _dotg(onehot_t, emb_ref[...], ((0,), (0,)))

    for i in range(NI):
        pre = _sp(jnp.dot(ea, w1_ref[i]) + b1_ref[i:i + 1, :])
        wt = jnp.dot(pre, w2_ref[i]) + b2_ref[i:i + 1, :]
        xf = jnp.dot(h, cw1_ref[i])
        wtc = (wt * cm_col).reshape(MB, MOL, MOL, NF)
        xf4 = xf.reshape(MB, MOL, NF)[:, None]
        agg = jnp.sum(wtc * xf4, axis=2).reshape(BLKA, NF)
        x2 = _sp(jnp.dot(agg, cw2_ref[i]) + cb2_ref[i:i + 1, :])
        x2 = jnp.dot(x2, bw_ref[i]) + bb_ref[i:i + 1, :]
        h = h + x2

    hl = _sp(jnp.dot(h, l1w_ref[...]) + l1b_ref[...])
    s1 = jnp.sum(hl * l2w_ref[...], axis=1, keepdims=True) + l2b_ref[...]
    zcol = _dotg(eye_ref[...], zrow.astype(f32), ((1,), (1,)))
    ho = jnp.where(zcol != 0.0, s1, 0.0)

    out_col = _dotg(sm_ref[...], ho, ((0,), (0,)))
    out_ref[0] = jnp.broadcast_to(out_col, (128, 128))


def kernel(pos, z, batch, emb, mlp_w1, mlp_b1, mlp_w2, mlp_b2,
           conv_w1, conv_w2, conv_b2, blk_w, blk_b,
           lin1_w, lin1_b, lin2_w, lin2_b):
    f32 = jnp.float32
    z3 = z.reshape(NBLK, 1, BLKA)
    r_e = jnp.arange(E_BLK, dtype=jnp.int32)[:, None]
    a_b = jnp.arange(BLKA, dtype=jnp.int32)[None, :]
    s_m = jnp.arange(MOL, dtype=jnp.int32)[None, :]
    l_128 = jnp.arange(128, dtype=jnp.int32)[None, :]
    af = (r_e // MOL == a_b).astype(f32)
    mf = (r_e % MOL == s_m).astype(f32)
    a_c = jnp.arange(BLKA, dtype=jnp.int32)[:, None]
    sm = (a_c // MOL == l_128).astype(f32)
    eye = (a_c == a_b).astype(f32)

    b2a = mlp_b2
    bba = blk_b
    l2ba = lin2_b

    full = lambda shape: pl.BlockSpec(shape, lambda b: (0,) * len(shape))
    out3 = pl.pallas_call(
        _schnet_kernel,
        grid=(NBLK,),
        in_specs=[
            pl.BlockSpec((BLKA, 3), lambda b: (b, 0)),
            pl.BlockSpec((1, 1, BLKA), lambda b: (b, 0, 0)),
            full((E_BLK, BLKA)), full((E_BLK, MOL)),
            full((BLKA, 128)), full((BLKA, BLKA)),
            full((100, NF)),
            full((NI, NG, NF)), full((NI, NF)),
            full((NI, NF, NF)), full((NI, NF)),
            full((NI, NF, NF)),
            full((NI, NF, NF)), full((NI, NF)),
            full((NI, NF, NF)), full((NI, NF)),
            full((NF, NF // 2)), full((1, NF // 2)),
            full((1, NF // 2)), full((1, 1)),
        ],
        out_specs=pl.BlockSpec((1, 128, 128), lambda b: (b, 0, 0)),
        out_shape=jax.ShapeDtypeStruct((NBLK, 128, 128), jnp.float32),
        compiler_params=pltpu.CompilerParams(
            dimension_semantics=("parallel",)),
    )(pos, z3, af, mf, sm, eye, emb, mlp_w1, mlp_b1, mlp_w2, b2a,
      conv_w1, conv_w2, conv_b2, blk_w, bba,
      lin1_w, lin1_b.reshape(1, -1), lin2_w.reshape(1, -1),
      l2ba.reshape(1, 1))
    return out3[:, :MB, 0].reshape(B)


# speedup vs baseline: 1.0252x; 1.0252x over previous
import jax
import jax.numpy as jnp
import numpy as np
from jax.experimental import pallas as pl
from jax.experimental.pallas import tpu as pltpu

N = 2048
MOL = 32
B = 64
K = 28
NF = 256
NG = 50
NI = 4
CUTOFF = 6.0

MB = 4
BLKA = MB * MOL
NBLK = N // BLKA
E_BLK = BLKA * MOL

_OFF_STEP = np.float32(CUTOFF / (NG - 1))
_COEFF = np.float32(-0.5 / _OFF_STEP ** 2)
_LOG2 = np.float32(np.log(2.0))
_NDROP = MOL - 1 - K


def _sp(x):
    return jax.nn.softplus(x) - _LOG2


def _dotg(a, b, dims):
    return jax.lax.dot_general(a, b, (dims, ((), ())))


def _iota(shape, dim):
    return jax.lax.broadcasted_iota(jnp.int32, shape, dim)


def _schnet_kernel(pos_ref, z_ref, mf_ref, sm_ref, eye_ref,
                   emb_ref, w1_ref, b1_ref, w2_ref, b2_ref,
                   cw1_ref, cw2_ref, cb2_ref, bw_ref, bb_ref,
                   l1w_ref, l1b_ref, l2w_ref, l2b_ref, out_ref):
    f32 = jnp.float32
    p = pos_ref[...]
    zrow = z_ref[0]

    ones3 = jnp.ones((MOL, 3), f32)
    ones1 = jnp.ones((MOL, 1), f32)
    ew_rows = []
    cm_rows = []
    for mm in range(MB):
        pm = p[mm * MOL:(mm + 1) * MOL, :]
        pp = pm * pm
        sq_col = _dotg(pp, ones3, ((1,), (1,)))
        sq_row = _dotg(ones3, pp, ((1,), (1,)))
        cross = _dotg(pm, pm, ((1,), (1,)))
        d2 = sq_col + sq_row - 2.0 * cross

        di = _iota((MOL, MOL), 0)
        si = _iota((MOL, MOL), 1)
        valid = di != si
        for _ in range(_NDROP):
            vals = jnp.where(valid, d2, -1e30)
            rmax = jnp.max(vals, axis=1, keepdims=True)
            cand = valid & (vals >= rmax)
            jmax = jnp.max(jnp.where(cand, si, -1), axis=1, keepdims=True)
            valid = valid & (si != jmax)

        acc = jnp.zeros((MOL, MOL), f32)
        for c in range(3):
            pc = pm[:, c:c + 1]
            pc_row = _dotg(ones1, pc, ((1,), (1,)))
            dif = pc - pc_row
            acc = acc + dif * dif
        ew = jnp.sqrt(acc)
        ccut = 0.5 * (jnp.cos(ew * np.pi / CUTOFF) + 1.0)
        cm_rows.append(jnp.where(valid, ccut, 0.0))
        ew_rows.append(ew)

    ew_all = jnp.concatenate(ew_rows, axis=0)
    cm_all = jnp.concatenate(cm_rows, axis=0)

    mf = mf_ref[...]
    ew_rep = jnp.broadcast_to(ew_all[:, None, :],
                              (BLKA, MOL, MOL)).reshape(E_BLK, MOL)
    ew_col = jnp.sum(ew_rep * mf, axis=1, keepdims=True)
    cm_rep = jnp.broadcast_to(cm_all[:, None, :],
                              (BLKA, MOL, MOL)).reshape(E_BLK, MOL)
    cm_col = jnp.sum(cm_rep * mf, axis=1, keepdims=True)

    off = (_iota((1, NG), 1).astype(f32)) * _OFF_STEP
    ea = jnp.exp(_COEFF * (ew_col - off) ** 2)

    onehot_t = (_iota((100, BLKA), 0) == zrow).astype(f32)
    h = _dotg(onehot_t, emb_ref[...], ((0,), (0,)))

    for i in range(NI):
        pre = _sp(jnp.dot(ea, w1_ref[i]) + b1_ref[i:i + 1, :])
        wt = jnp.dot(pre, w2_ref[i]) + b2_ref[i:i + 1, :]
        xf = jnp.dot(h, cw1_ref[i])
        wtc = (wt * cm_col).reshape(MB, MOL, MOL, NF)
        xf4 = xf.reshape(MB, MOL, NF)[:, None]
        agg = jnp.sum(wtc * xf4, axis=2).reshape(BLKA, NF)
        x2 = _sp(jnp.dot(agg, cw2_ref[i]) + cb2_ref[i:i + 1, :])
        x2 = jnp.dot(x2, bw_ref[i]) + bb_ref[i:i + 1, :]
        h = h + x2

    hl = _sp(jnp.dot(h, l1w_ref[...]) + l1b_ref[...])
    s1 = jnp.sum(hl * l2w_ref[...], axis=1, keepdims=True) + l2b_ref[...]
    zcol = _dotg(eye_ref[...], zrow.astype(f32), ((1,), (1,)))
    ho = jnp.where(zcol != 0.0, s1, 0.0)

    out_col = _dotg(sm_ref[...], ho, ((0,), (0,)))
    out_ref[0] = jnp.broadcast_to(out_col, (128, 128))


def kernel(pos, z, batch, emb, mlp_w1, mlp_b1, mlp_w2, mlp_b2,
           conv_w1, conv_w2, conv_b2, blk_w, blk_b,
           lin1_w, lin1_b, lin2_w, lin2_b):
    f32 = jnp.float32
    z3 = z.reshape(NBLK, 1, BLKA)
    r_e = jnp.arange(E_BLK, dtype=jnp.int32)[:, None]
    a_b = jnp.arange(BLKA, dtype=jnp.int32)[None, :]
    s_m = jnp.arange(MOL, dtype=jnp.int32)[None, :]
    l_128 = jnp.arange(128, dtype=jnp.int32)[None, :]
    mf = (r_e % MOL == s_m).astype(f32)
    a_c = jnp.arange(BLKA, dtype=jnp.int32)[:, None]
    sm = (a_c // MOL == l_128).astype(f32)
    eye = (a_c == a_b).astype(f32)

    b2a = mlp_b2
    bba = blk_b
    l2ba = lin2_b

    full = lambda shape: pl.BlockSpec(shape, lambda b: (0,) * len(shape))
    out3 = pl.pallas_call(
        _schnet_kernel,
        grid=(NBLK,),
        in_specs=[
            pl.BlockSpec((BLKA, 3), lambda b: (b, 0)),
            pl.BlockSpec((1, 1, BLKA), lambda b: (b, 0, 0)),
            full((E_BLK, MOL)),
            full((BLKA, 128)), full((BLKA, BLKA)),
            full((100, NF)),
            full((NI, NG, NF)), full((NI, NF)),
            full((NI, NF, NF)), full((NI, NF)),
            full((NI, NF, NF)),
            full((NI, NF, NF)), full((NI, NF)),
            full((NI, NF, NF)), full((NI, NF)),
            full((NF, NF // 2)), full((1, NF // 2)),
            full((1, NF // 2)), full((1, 1)),
        ],
        out_specs=pl.BlockSpec((1, 128, 128), lambda b: (b, 0, 0)),
        out_shape=jax.ShapeDtypeStruct((NBLK, 128, 128), jnp.float32),
        compiler_params=pltpu.CompilerParams(
            dimension_semantics=("parallel",)),
    )(pos, z3, mf, sm, eye, emb, mlp_w1, mlp_b1, mlp_w2, b2a,
      conv_w1, conv_w2, conv_b2, blk_w, bba,
      lin1_w, lin1_b.reshape(1, -1), lin2_w.reshape(1, -1),
      l2ba.reshape(1, 1))
    return out3[:, :MB, 0].reshape(B)
